# speedup vs baseline: 1.7578x; 1.7578x over previous
import functools

import jax
import jax.numpy as jnp
from jax.experimental import pallas as pl
from jax.experimental.pallas import tpu as pltpu

H = 256
NP = 10240
MB = 2048


def _mlp_body(a_ref, h_ref, agg_ref, w1_ref, c1_ref, w2_ref, c2_ref, o_ref):
    z = a_ref[0] * h_ref[...] + agg_ref[...]
    u = jnp.dot(z, w1_ref[...], preferred_element_type=jnp.float32) + c1_ref[...]
    u = jnp.maximum(u, 0.0)
    v = jnp.dot(u, w2_ref[...], preferred_element_type=jnp.float32) + c2_ref[...]
    o_ref[...] = jnp.maximum(v, 0.0)


@jax.jit
def _mlp(a, h, agg, w1, c1, w2, c2):
    grid = (NP // MB,)
    return pl.pallas_call(
        _mlp_body,
        grid=grid,
        in_specs=[
            pl.BlockSpec(memory_space=pltpu.SMEM),
            pl.BlockSpec((MB, H), lambda i: (i, 0)),
            pl.BlockSpec((MB, H), lambda i: (i, 0)),
            pl.BlockSpec((H, 2 * H), lambda i: (0, 0)),
            pl.BlockSpec((1, 2 * H), lambda i: (0, 0)),
            pl.BlockSpec((2 * H, H), lambda i: (0, 0)),
            pl.BlockSpec((1, H), lambda i: (0, 0)),
        ],
        out_specs=pl.BlockSpec((MB, H), lambda i: (i, 0)),
        out_shape=jax.ShapeDtypeStruct((NP, H), jnp.float32),
    )(a, h, agg, w1, c1, w2, c2)


def kernel(x, edge_index, edge_attr, batch, atom_emb, bond_emb, eps, W1, b1, g1, be1, W2, b2, g2, be2, Wp, bp):
    n = x.shape[0]
    src = edge_index[0]
    dst = edge_index[1]

    W1f = W1 * g1[:, None, :]
    c1 = (b1 * g1 + be1)[:, None, :]
    W2f = W2 * g2[:, None, :]
    c2 = (b2 * g2 + be2)[:, None, :]

    h = jnp.zeros((n, H), jnp.float32)
    for f in range(9):
        h = h + atom_emb[f][x[:, f]]
    hp = jnp.zeros((NP, H), jnp.float32).at[:n].set(h)

    bt = (bond_emb[:, 0, :, None, None, None, :]
          + bond_emb[:, 1, None, :, None, None, :]
          + bond_emb[:, 2, None, None, :, None, :]
          + bond_emb[:, 3, None, None, None, :, :]).reshape(4, 4096, H)
    comb = edge_attr[:, 0] + 8 * edge_attr[:, 1] + 64 * edge_attr[:, 2] + 512 * edge_attr[:, 3]

    for i in range(4):
        e = bt[i][comb]
        msg = jax.nn.relu(hp[src] + e)
        agg = jnp.zeros((NP, H), jnp.float32).at[dst].add(msg)
        hp = _mlp(1.0 + eps[i:i + 1], hp, agg, W1f[i], c1[i], W2f[i], c2[i])

    nr = hp[:n]
    sums = jax.ops.segment_sum(nr, batch, num_segments=64)
    cnt = jax.ops.segment_sum(jnp.ones((n,), jnp.float32), batch, num_segments=64)
    hg = sums / jnp.maximum(cnt, 1.0)[:, None]
    return hg @ Wp + bp
